# baseline (speedup 1.0000x reference)
import functools
import math

import jax
import jax.numpy as jnp
from jax import lax
from jax.experimental import pallas as pl
from jax.experimental.pallas import tpu as pltpu
from jax.experimental.pallas import tpu_sc as plsc

_NC = 2
_NS = 16
_NW = _NC * _NS
_L = 16

_CHUNK = 80


def _qkv_body(x_ref, wq_ref, wk_ref, wv_ref, bq_ref, bk_ref, bv_ref,
              q_ref, k_ref, v_ref):
    xb = x_ref[...]
    dn = (((1,), (1,)), ((), ()))
    q_ref[...] = lax.dot_general(xb, wq_ref[...], dn,
                                 preferred_element_type=jnp.float32) + bq_ref[...]
    k_ref[...] = lax.dot_general(xb, wk_ref[...], dn,
                                 preferred_element_type=jnp.float32) + bk_ref[...]
    v_ref[...] = lax.dot_general(xb, wv_ref[...], dn,
                                 preferred_element_type=jnp.float32) + bv_ref[...]


def _qkv(x, Wq, Wk, Wv, bq, bk, bv):
    n, d = x.shape
    blk = 2000
    grid = n // blk
    row_spec = pl.BlockSpec((blk, d), lambda i: (i, 0))
    w_spec = pl.BlockSpec((d, d), lambda i: (0, 0))
    b_spec = pl.BlockSpec((1, d), lambda i: (0, 0))
    out = jax.ShapeDtypeStruct((n, d), jnp.float32)
    return pl.pallas_call(
        _qkv_body,
        grid=(grid,),
        in_specs=[row_spec, w_spec, w_spec, w_spec, b_spec, b_spec, b_spec],
        out_specs=[row_spec, row_spec, row_spec],
        out_shape=[out, out, out],
    )(x, Wq, Wk, Wv, bq.reshape(1, d), bk.reshape(1, d), bv.reshape(1, d))


def _make_energy(n, e, d):
    epw = e // _NW
    nchunk = epw // _CHUNK
    inv_scale = 1.0 / math.sqrt(d)
    mesh = plsc.VectorSubcoreMesh(core_axis_name="c", subcore_axis_name="s")

    @functools.partial(
        pl.kernel,
        out_type=jax.ShapeDtypeStruct((e,), jnp.float32),
        mesh=mesh,
        scratch_types=[
            pltpu.VMEM((2 * 3 * _CHUNK,), jnp.int32),
            pltpu.VMEM((2, _CHUNK, d), jnp.float32),
            pltpu.VMEM((2, _CHUNK, d), jnp.float32),
            pltpu.VMEM((epw,), jnp.float32),
            pltpu.SemaphoreType.DMA,
            pltpu.SemaphoreType.DMA,
            pltpu.SemaphoreType.DMA,
        ],
        compiler_params=pltpu.CompilerParams(needs_layout_passes=False),
    )
    def energy_kernel(q_hbm, k_hbm, aux_hbm, z_hbm,
                      auxb, qbuf, kbuf, zv, semq, semk, sema):
        cid = lax.axis_index("c")
        sid = lax.axis_index("s")
        wid = sid * _NC + cid
        cbase = wid * nchunk

        aw = 3 * _CHUNK

        def fire_aux(c, slot):
            pltpu.async_copy(aux_hbm.at[pl.ds((cbase + c) * aw, aw)],
                             auxb.at[pl.ds(slot * aw, aw)], sema)

        def wait_aux(slot):
            del slot
            pltpu.make_async_copy(aux_hbm.at[pl.ds(0, aw)],
                                  auxb.at[pl.ds(0, aw)], sema).wait()

        def fire_rows(c, slot):
            pltpu.async_copy(
                q_hbm.at[auxb.at[pl.ds(slot * aw, _CHUNK)]], qbuf.at[slot],
                semq)
            pltpu.async_copy(
                k_hbm.at[auxb.at[pl.ds(slot * aw + _CHUNK, _CHUNK)]],
                kbuf.at[slot], semk)

        def wait_rows(slot):
            pltpu.make_async_copy(q_hbm.at[pl.ds(0, _CHUNK)], qbuf.at[slot],
                                  semq).wait()
            pltpu.make_async_copy(k_hbm.at[pl.ds(0, _CHUNK)], kbuf.at[slot],
                                  semk).wait()

        pltpu.sync_copy(aux_hbm.at[pl.ds(cbase * aw, aw)],
                        auxb.at[pl.ds(0, aw)])
        fire_rows(0, 0)
        fire_aux(1, 1)
        lanes = lax.iota(jnp.int32, _L)

        def chunk_body(ci, carry):
            slot = lax.rem(ci, 2)
            nslot = 1 - slot
            wait_rows(slot)

            @pl.when(ci + 1 < nchunk)
            def _():
                wait_aux(nslot)
                fire_rows(ci + 1, nslot)

            slots = jnp.full((_L,), slot, jnp.int32)
            for g in range(_CHUNK // _L):
                rows = lanes + g * _L
                accs = []
                for m in range(4):
                    cols = jnp.full((_L,), m, jnp.int32)
                    accs.append(plsc.load_gather(qbuf, [slots, rows, cols]) *
                                plsc.load_gather(kbuf, [slots, rows, cols]))
                for j in range(4, d):
                    cols = jnp.full((_L,), j, jnp.int32)
                    accs[j % 4] = accs[j % 4] + (
                        plsc.load_gather(qbuf, [slots, rows, cols]) *
                        plsc.load_gather(kbuf, [slots, rows, cols]))
                ev = (accs[0] + accs[1]) + (accs[2] + accs[3])
                ew = plsc.bitcast(
                    auxb[pl.ds(slot * aw + 2 * _CHUNK + g * _L, _L)],
                    jnp.float32)
                zv[pl.ds(ci * _CHUNK + g * _L, _L)] = ev * ew * inv_scale

            @pl.when(ci + 2 < nchunk)
            def _():
                fire_aux(ci + 2, slot)

            return carry

        lax.fori_loop(0, nchunk, chunk_body, 0)
        pltpu.sync_copy(zv, z_hbm.at[pl.ds(wid * epw, epw)])

    return energy_kernel


def _softmax_body(z_ref, a_ref):
    z = z_ref[...]
    m = jnp.max(z)
    p = jnp.exp(z - m)
    a_ref[...] = p / jnp.sum(p)


def _softmax(z2d):
    return pl.pallas_call(
        _softmax_body,
        out_shape=jax.ShapeDtypeStruct(z2d.shape, jnp.float32),
    )(z2d)


def _make_scatter(n, e, d):
    epw = e // _NW
    nchunk = epw // _CHUNK
    zrows = 40
    ncopy_total = n // zrows
    ncopy_iters = -(-ncopy_total // _NS)
    orows = 200
    nout_total = n // orows
    nout_iters = -(-nout_total // _NS)
    mesh = plsc.VectorSubcoreMesh(core_axis_name="c", subcore_axis_name="s")

    @functools.partial(
        pl.kernel,
        out_type=jax.ShapeDtypeStruct((_NC, n, d), jnp.float32),
        mesh=mesh,
        scratch_types=[
            pltpu.VMEM((2 * 3 * _CHUNK,), jnp.int32),
            pltpu.VMEM((2 * _CHUNK,), jnp.float32),
            pltpu.VMEM((_CHUNK,), jnp.int32),
            pltpu.VMEM((2, _CHUNK, d), jnp.float32),
            pltpu.VMEM((zrows, d), jnp.float32),
            pltpu.VMEM_SHARED((n, d), jnp.float32),
            pltpu.SemaphoreType.DMA,
            pltpu.SemaphoreType.DMA,
            pltpu.SemaphoreType.DMA,
            pltpu.SemaphoreType.DMA,
        ],
        compiler_params=pltpu.CompilerParams(needs_layout_passes=False),
    )
    def scatter_kernel(v_hbm, aux_hbm, attn_hbm, out_hbm,
                       auxb, abuf, rc, vbuf, zbuf, acc_sh,
                       semv, sems, sema, semw):
        cid = lax.axis_index("c")
        sid = lax.axis_index("s")
        wid = sid * _NC + cid
        cbase = wid * nchunk
        ebase = wid * epw

        def zrow_body(r, carry):
            for j in range(d // _L):
                zbuf[r, pl.ds(j * _L, _L)] = jnp.zeros((_L,), jnp.float32)
            return carry

        lax.fori_loop(0, zrows, zrow_body, 0)

        def zcopy_body(t, carry):
            blk = t * _NS + sid

            @pl.when(blk < ncopy_total)
            def _():
                pltpu.sync_copy(zbuf, acc_sh.at[pl.ds(blk * zrows, zrows)])

            return carry

        lax.fori_loop(0, ncopy_iters, zcopy_body, 0)
        plsc.subcore_barrier()

        aw = 3 * _CHUNK

        def fire_aux(c, slot):
            pltpu.async_copy(aux_hbm.at[pl.ds((cbase + c) * aw, aw)],
                             auxb.at[pl.ds(slot * aw, aw)], sema)
            pltpu.async_copy(attn_hbm.at[pl.ds(ebase + c * _CHUNK, _CHUNK)],
                             abuf.at[pl.ds(slot * _CHUNK, _CHUNK)], semw)

        def wait_aux(slot):
            del slot
            pltpu.make_async_copy(aux_hbm.at[pl.ds(0, aw)],
                                  auxb.at[pl.ds(0, aw)], sema).wait()
            pltpu.make_async_copy(attn_hbm.at[pl.ds(0, _CHUNK)],
                                  abuf.at[pl.ds(0, _CHUNK)], semw).wait()

        def fire_v(c, slot):
            pltpu.async_copy(
                v_hbm.at[auxb.at[pl.ds(slot * aw + _CHUNK, _CHUNK)]],
                vbuf.at[slot], semv)

        def wait_v(slot):
            pltpu.make_async_copy(v_hbm.at[pl.ds(0, _CHUNK)], vbuf.at[slot],
                                  semv).wait()

        def drain_scatter(slot):
            pltpu.make_async_copy(vbuf.at[slot], acc_sh.at[rc], sems).wait()

        pltpu.sync_copy(aux_hbm.at[pl.ds(cbase * aw, aw)],
                        auxb.at[pl.ds(0, aw)])
        pltpu.sync_copy(attn_hbm.at[pl.ds(ebase, _CHUNK)],
                        abuf.at[pl.ds(0, _CHUNK)])
        fire_v(0, 0)
        fire_aux(1, 1)

        def chunk_body(ci, carry):
            slot = lax.rem(ci, 2)
            nslot = 1 - slot
            wait_v(slot)

            @pl.when(ci > 0)
            def _():
                drain_scatter(nslot)

            @pl.when(ci + 1 < nchunk)
            def _():
                wait_aux(nslot)
                fire_v(ci + 1, nslot)

            vc = vbuf.at[slot]
            for g in range(_CHUNK // _L):
                off = g * _L
                rc[pl.ds(off, _L)] = auxb[pl.ds(slot * aw + off, _L)]
                a16 = abuf[pl.ds(slot * _CHUNK + off, _L)]
                for i in range(_L):
                    ei = off + i
                    a = a16[i]
                    for j in range(d // _L):
                        vc[ei, pl.ds(j * _L, _L)] = (
                            vc[ei, pl.ds(j * _L, _L)] * a)
            pltpu.async_copy(vbuf.at[slot], acc_sh.at[rc], sems, add=True)

            @pl.when(ci + 2 < nchunk)
            def _():
                fire_aux(ci + 2, slot)

            return carry

        lax.fori_loop(0, nchunk, chunk_body, 0)
        drain_scatter((nchunk - 1) % 2)
        plsc.subcore_barrier()

        def ocopy_body(t, carry):
            blk = t * _NS + sid

            @pl.when(blk < nout_total)
            def _():
                r0 = blk * orows
                pltpu.sync_copy(acc_sh.at[pl.ds(r0, orows)],
                                out_hbm.at[cid, pl.ds(r0, orows)])

            return carry

        lax.fori_loop(0, nout_iters, ocopy_body, 0)

    return scatter_kernel


def _combine_body(p_ref, o_ref):
    o_ref[...] = p_ref[0] + p_ref[1]


def _combine(part):
    _, n, d = part.shape
    blk = 2000
    return pl.pallas_call(
        _combine_body,
        grid=(n // blk,),
        in_specs=[pl.BlockSpec((2, blk, d), lambda i: (0, i, 0))],
        out_specs=pl.BlockSpec((blk, d), lambda i: (i, 0)),
        out_shape=jax.ShapeDtypeStruct((n, d), jnp.float32),
    )(part)


def kernel(x, edge_index, edge_weight, Wq, bq, Wk, bk, Wv, bv):
    n, d = x.shape
    e = edge_weight.shape[0]

    ew_bits = lax.bitcast_convert_type(edge_weight, jnp.int32)
    aux = jnp.stack([edge_index[0], edge_index[1], ew_bits], axis=0)
    aux3 = aux.reshape(3, e // _CHUNK, _CHUNK).transpose(1, 0, 2)
    aux3 = aux3.reshape(e // _CHUNK * 3 * _CHUNK)

    q, k, v = _qkv(x, Wq, Wk, Wv, bq, bk, bv)
    z = _make_energy(n, e, d)(q, k, aux3)
    attn = _softmax(z.reshape(e // 128, 128)).reshape(e)
    part = _make_scatter(n, e, d)(v, aux3, attn)
    return _combine(part)


# speedup vs baseline: 2.5755x; 2.5755x over previous
import functools
import math

import jax
import jax.numpy as jnp
from jax import lax
from jax.experimental import pallas as pl
from jax.experimental.pallas import tpu as pltpu
from jax.experimental.pallas import tpu_sc as plsc

_NC = 2
_NS = 16
_NW = _NC * _NS
_L = 16

_CHUNK = 80

_GD = lax.GatherDimensionNumbers(offset_dims=(), collapsed_slice_dims=(0,),
                                 start_index_map=(0,))


def _shuffle(x, pm):
    return lax.gather(x, pm[:, None], _GD, slice_sizes=(1,),
                      mode=lax.GatherScatterMode.PROMISE_IN_BOUNDS)


def _qkv_body(x_ref, wq_ref, wk_ref, wv_ref, bq_ref, bk_ref, bv_ref,
              q_ref, k_ref, v_ref):
    xb = x_ref[...]
    dn = (((1,), (1,)), ((), ()))
    q_ref[...] = lax.dot_general(xb, wq_ref[...], dn,
                                 preferred_element_type=jnp.float32) + bq_ref[...]
    k_ref[...] = lax.dot_general(xb, wk_ref[...], dn,
                                 preferred_element_type=jnp.float32) + bk_ref[...]
    v_ref[...] = lax.dot_general(xb, wv_ref[...], dn,
                                 preferred_element_type=jnp.float32) + bv_ref[...]


def _qkv(x, Wq, Wk, Wv, bq, bk, bv):
    n, d = x.shape
    blk = 2000
    grid = n // blk
    row_spec = pl.BlockSpec((blk, d), lambda i: (i, 0))
    w_spec = pl.BlockSpec((d, d), lambda i: (0, 0))
    b_spec = pl.BlockSpec((1, d), lambda i: (0, 0))
    out = jax.ShapeDtypeStruct((n, d), jnp.float32)
    return pl.pallas_call(
        _qkv_body,
        grid=(grid,),
        in_specs=[row_spec, w_spec, w_spec, w_spec, b_spec, b_spec, b_spec],
        out_specs=[row_spec, row_spec, row_spec],
        out_shape=[out, out, out],
    )(x, Wq, Wk, Wv, bq.reshape(1, d), bk.reshape(1, d), bv.reshape(1, d))


def _make_energy(n, e, d):
    epw = e // _NW
    nchunk = epw // _CHUNK
    inv_scale = 1.0 / math.sqrt(d)
    mesh = plsc.VectorSubcoreMesh(core_axis_name="c", subcore_axis_name="s")

    @functools.partial(
        pl.kernel,
        out_type=jax.ShapeDtypeStruct((e,), jnp.float32),
        mesh=mesh,
        scratch_types=[
            pltpu.VMEM((2 * 3 * _CHUNK,), jnp.int32),
            pltpu.VMEM((2, _CHUNK, d), jnp.float32),
            pltpu.VMEM((2, _CHUNK, d), jnp.float32),
            pltpu.VMEM((epw,), jnp.float32),
            pltpu.SemaphoreType.DMA,
            pltpu.SemaphoreType.DMA,
            pltpu.SemaphoreType.DMA,
        ],
        compiler_params=pltpu.CompilerParams(needs_layout_passes=False),
    )
    def energy_kernel(q_hbm, k_hbm, aux_hbm, z_hbm,
                      auxb, qbuf, kbuf, zv, semq, semk, sema):
        cid = lax.axis_index("c")
        sid = lax.axis_index("s")
        wid = sid * _NC + cid
        cbase = wid * nchunk

        aw = 3 * _CHUNK

        def fire_aux(c, slot):
            pltpu.async_copy(aux_hbm.at[pl.ds((cbase + c) * aw, aw)],
                             auxb.at[pl.ds(slot * aw, aw)], sema)

        def wait_aux(slot):
            del slot
            pltpu.make_async_copy(aux_hbm.at[pl.ds(0, aw)],
                                  auxb.at[pl.ds(0, aw)], sema).wait()

        def fire_rows(c, slot):
            pltpu.async_copy(
                q_hbm.at[auxb.at[pl.ds(slot * aw, _CHUNK)]], qbuf.at[slot],
                semq)
            pltpu.async_copy(
                k_hbm.at[auxb.at[pl.ds(slot * aw + _CHUNK, _CHUNK)]],
                kbuf.at[slot], semk)

        def wait_rows(slot):
            pltpu.make_async_copy(q_hbm.at[pl.ds(0, _CHUNK)], qbuf.at[slot],
                                  semq).wait()
            pltpu.make_async_copy(k_hbm.at[pl.ds(0, _CHUNK)], kbuf.at[slot],
                                  semk).wait()

        pltpu.sync_copy(aux_hbm.at[pl.ds(cbase * aw, aw)],
                        auxb.at[pl.ds(0, aw)])
        fire_rows(0, 0)
        fire_aux(1, 1)
        lanes = lax.iota(jnp.int32, _L)

        def chunk_body(ci, carry):
            slot = lax.rem(ci, 2)
            nslot = 1 - slot
            wait_rows(slot)

            @pl.when(ci + 1 < nchunk)
            def _():
                wait_aux(nslot)
                fire_rows(ci + 1, nslot)

            perms = [lanes ^ s for s in (1, 2, 4, 8)]
            for g in range(_CHUNK // _L):
                ev = jnp.zeros((_L,), jnp.float32)
                for i in range(_L):
                    ei_ = g * _L + i
                    acc = (qbuf[slot, ei_, pl.ds(0, _L)] *
                           kbuf[slot, ei_, pl.ds(0, _L)])
                    for j in range(1, d // _L):
                        acc = acc + (qbuf[slot, ei_, pl.ds(j * _L, _L)] *
                                     kbuf[slot, ei_, pl.ds(j * _L, _L)])
                    for pm in perms:
                        acc = acc + _shuffle(acc, pm)
                    ev = jnp.where(lanes == i, acc, ev)
                ew = plsc.bitcast(
                    auxb[pl.ds(slot * aw + 2 * _CHUNK + g * _L, _L)],
                    jnp.float32)
                zv[pl.ds(ci * _CHUNK + g * _L, _L)] = ev * ew * inv_scale

            @pl.when(ci + 2 < nchunk)
            def _():
                fire_aux(ci + 2, slot)

            return carry

        lax.fori_loop(0, nchunk, chunk_body, 0)
        pltpu.sync_copy(zv, z_hbm.at[pl.ds(wid * epw, epw)])

    return energy_kernel


def _softmax_body(z_ref, a_ref):
    z = z_ref[...]
    m = jnp.max(z)
    p = jnp.exp(z - m)
    a_ref[...] = p / jnp.sum(p)


def _softmax(z2d):
    return pl.pallas_call(
        _softmax_body,
        out_shape=jax.ShapeDtypeStruct(z2d.shape, jnp.float32),
    )(z2d)


def _make_scatter(n, e, d):
    epw = e // _NW
    nchunk = epw // _CHUNK
    zrows = 40
    ncopy_total = n // zrows
    ncopy_iters = -(-ncopy_total // _NS)
    orows = 200
    nout_total = n // orows
    nout_iters = -(-nout_total // _NS)
    mesh = plsc.VectorSubcoreMesh(core_axis_name="c", subcore_axis_name="s")

    @functools.partial(
        pl.kernel,
        out_type=jax.ShapeDtypeStruct((_NC, n, d), jnp.float32),
        mesh=mesh,
        scratch_types=[
            pltpu.VMEM((2 * 3 * _CHUNK,), jnp.int32),
            pltpu.VMEM((2 * _CHUNK,), jnp.float32),
            pltpu.VMEM((_CHUNK,), jnp.int32),
            pltpu.VMEM((2, _CHUNK, d), jnp.float32),
            pltpu.VMEM((zrows, d), jnp.float32),
            pltpu.VMEM_SHARED((n, d), jnp.float32),
            pltpu.SemaphoreType.DMA,
            pltpu.SemaphoreType.DMA,
            pltpu.SemaphoreType.DMA,
            pltpu.SemaphoreType.DMA,
        ],
        compiler_params=pltpu.CompilerParams(needs_layout_passes=False),
    )
    def scatter_kernel(v_hbm, aux_hbm, attn_hbm, out_hbm,
                       auxb, abuf, rc, vbuf, zbuf, acc_sh,
                       semv, sems, sema, semw):
        cid = lax.axis_index("c")
        sid = lax.axis_index("s")
        wid = sid * _NC + cid
        cbase = wid * nchunk
        ebase = wid * epw

        def zrow_body(r, carry):
            for j in range(d // _L):
                zbuf[r, pl.ds(j * _L, _L)] = jnp.zeros((_L,), jnp.float32)
            return carry

        lax.fori_loop(0, zrows, zrow_body, 0)

        def zcopy_body(t, carry):
            blk = t * _NS + sid

            @pl.when(blk < ncopy_total)
            def _():
                pltpu.sync_copy(zbuf, acc_sh.at[pl.ds(blk * zrows, zrows)])

            return carry

        lax.fori_loop(0, ncopy_iters, zcopy_body, 0)
        plsc.subcore_barrier()

        aw = 3 * _CHUNK

        def fire_aux(c, slot):
            pltpu.async_copy(aux_hbm.at[pl.ds((cbase + c) * aw, aw)],
                             auxb.at[pl.ds(slot * aw, aw)], sema)
            pltpu.async_copy(attn_hbm.at[pl.ds(ebase + c * _CHUNK, _CHUNK)],
                             abuf.at[pl.ds(slot * _CHUNK, _CHUNK)], semw)

        def wait_aux(slot):
            del slot
            pltpu.make_async_copy(aux_hbm.at[pl.ds(0, aw)],
                                  auxb.at[pl.ds(0, aw)], sema).wait()
            pltpu.make_async_copy(attn_hbm.at[pl.ds(0, _CHUNK)],
                                  abuf.at[pl.ds(0, _CHUNK)], semw).wait()

        def fire_v(c, slot):
            pltpu.async_copy(
                v_hbm.at[auxb.at[pl.ds(slot * aw + _CHUNK, _CHUNK)]],
                vbuf.at[slot], semv)

        def wait_v(slot):
            pltpu.make_async_copy(v_hbm.at[pl.ds(0, _CHUNK)], vbuf.at[slot],
                                  semv).wait()

        def drain_scatter(slot):
            pltpu.make_async_copy(vbuf.at[slot], acc_sh.at[rc], sems).wait()

        pltpu.sync_copy(aux_hbm.at[pl.ds(cbase * aw, aw)],
                        auxb.at[pl.ds(0, aw)])
        pltpu.sync_copy(attn_hbm.at[pl.ds(ebase, _CHUNK)],
                        abuf.at[pl.ds(0, _CHUNK)])
        fire_v(0, 0)
        fire_aux(1, 1)

        def chunk_body(ci, carry):
            slot = lax.rem(ci, 2)
            nslot = 1 - slot
            wait_v(slot)

            @pl.when(ci > 0)
            def _():
                drain_scatter(nslot)

            @pl.when(ci + 1 < nchunk)
            def _():
                wait_aux(nslot)
                fire_v(ci + 1, nslot)

            vc = vbuf.at[slot]
            for g in range(_CHUNK // _L):
                off = g * _L
                rc[pl.ds(off, _L)] = auxb[pl.ds(slot * aw + off, _L)]
                a16 = abuf[pl.ds(slot * _CHUNK + off, _L)]
                for i in range(_L):
                    ei = off + i
                    a = a16[i]
                    for j in range(d // _L):
                        vc[ei, pl.ds(j * _L, _L)] = (
                            vc[ei, pl.ds(j * _L, _L)] * a)
            pltpu.async_copy(vbuf.at[slot], acc_sh.at[rc], sems, add=True)

            @pl.when(ci + 2 < nchunk)
            def _():
                fire_aux(ci + 2, slot)

            return carry

        lax.fori_loop(0, nchunk, chunk_body, 0)
        drain_scatter((nchunk - 1) % 2)
        plsc.subcore_barrier()

        def ocopy_body(t, carry):
            blk = t * _NS + sid

            @pl.when(blk < nout_total)
            def _():
                r0 = blk * orows
                pltpu.sync_copy(acc_sh.at[pl.ds(r0, orows)],
                                out_hbm.at[cid, pl.ds(r0, orows)])

            return carry

        lax.fori_loop(0, nout_iters, ocopy_body, 0)

    return scatter_kernel


def _combine_body(p_ref, o_ref):
    o_ref[...] = p_ref[0] + p_ref[1]


def _combine(part):
    _, n, d = part.shape
    blk = 2000
    return pl.pallas_call(
        _combine_body,
        grid=(n // blk,),
        in_specs=[pl.BlockSpec((2, blk, d), lambda i: (0, i, 0))],
        out_specs=pl.BlockSpec((blk, d), lambda i: (i, 0)),
        out_shape=jax.ShapeDtypeStruct((n, d), jnp.float32),
    )(part)


def kernel(x, edge_index, edge_weight, Wq, bq, Wk, bk, Wv, bv):
    n, d = x.shape
    e = edge_weight.shape[0]

    ew_bits = lax.bitcast_convert_type(edge_weight, jnp.int32)
    aux = jnp.stack([edge_index[0], edge_index[1], ew_bits], axis=0)
    aux3 = aux.reshape(3, e // _CHUNK, _CHUNK).transpose(1, 0, 2)
    aux3 = aux3.reshape(e // _CHUNK * 3 * _CHUNK)

    q, k, v = _qkv(x, Wq, Wk, Wv, bq, bk, bv)
    z = _make_energy(n, e, d)(q, k, aux3)
    attn = _softmax(z.reshape(e // 128, 128)).reshape(e)
    part = _make_scatter(n, e, d)(v, aux3, attn)
    return _combine(part)


# speedup vs baseline: 3.8582x; 1.4980x over previous
import functools
import math

import jax
import jax.numpy as jnp
from jax import lax
from jax.experimental import pallas as pl
from jax.experimental.pallas import tpu as pltpu
from jax.experimental.pallas import tpu_sc as plsc

_NC = 2
_NS = 16
_NW = _NC * _NS
_L = 16

_CHUNK = 80

_GD = lax.GatherDimensionNumbers(offset_dims=(), collapsed_slice_dims=(0,),
                                 start_index_map=(0,))


def _shuffle(x, pm):
    return lax.gather(x, pm[:, None], _GD, slice_sizes=(1,),
                      mode=lax.GatherScatterMode.PROMISE_IN_BOUNDS)


def _qkv_body(x_ref, wq_ref, wk_ref, wv_ref, bq_ref, bk_ref, bv_ref,
              q_ref, k_ref, v_ref):
    xb = x_ref[...]
    dn = (((1,), (1,)), ((), ()))
    q = lax.dot_general(xb, wq_ref[...], dn,
                        preferred_element_type=jnp.float32) + bq_ref[...]
    q_ref[...] = q.astype(jnp.bfloat16)
    k = lax.dot_general(xb, wk_ref[...], dn,
                        preferred_element_type=jnp.float32) + bk_ref[...]
    k_ref[...] = k.astype(jnp.bfloat16)
    v_ref[...] = lax.dot_general(xb, wv_ref[...], dn,
                                 preferred_element_type=jnp.float32) + bv_ref[...]


def _qkv(x, Wq, Wk, Wv, bq, bk, bv):
    n, d = x.shape
    blk = 2000
    grid = n // blk
    row_spec = pl.BlockSpec((blk, d), lambda i: (i, 0))
    w_spec = pl.BlockSpec((d, d), lambda i: (0, 0))
    b_spec = pl.BlockSpec((1, d), lambda i: (0, 0))
    outb = jax.ShapeDtypeStruct((n, d), jnp.bfloat16)
    outf = jax.ShapeDtypeStruct((n, d), jnp.float32)
    return pl.pallas_call(
        _qkv_body,
        grid=(grid,),
        in_specs=[row_spec, w_spec, w_spec, w_spec, b_spec, b_spec, b_spec],
        out_specs=[row_spec, row_spec, row_spec],
        out_shape=[outb, outb, outf],
    )(x, Wq, Wk, Wv, bq.reshape(1, d), bk.reshape(1, d), bv.reshape(1, d))


def _make_energy(n, e, d):
    epw = e // _NW
    nchunk = epw // _CHUNK
    inv_scale = 1.0 / math.sqrt(d)
    mesh = plsc.VectorSubcoreMesh(core_axis_name="c", subcore_axis_name="s")

    @functools.partial(
        pl.kernel,
        out_type=jax.ShapeDtypeStruct((e,), jnp.float32),
        mesh=mesh,
        scratch_types=[
            pltpu.VMEM((2 * 3 * _CHUNK,), jnp.int32),
            pltpu.VMEM((2, _CHUNK, d // 2), jnp.int32),
            pltpu.VMEM((2, _CHUNK, d // 2), jnp.int32),
            pltpu.VMEM((epw,), jnp.float32),
            pltpu.SemaphoreType.DMA,
            pltpu.SemaphoreType.DMA,
            pltpu.SemaphoreType.DMA,
        ],
        compiler_params=pltpu.CompilerParams(needs_layout_passes=False,
                                             use_tc_tiling_on_sc=False),
    )
    def energy_kernel(q_hbm, k_hbm, aux_hbm, z_hbm,
                      auxb, qbuf, kbuf, zv, semq, semk, sema):
        cid = lax.axis_index("c")
        sid = lax.axis_index("s")
        wid = sid * _NC + cid
        cbase = wid * nchunk

        aw = 3 * _CHUNK

        def fire_aux(c, slot):
            pltpu.async_copy(aux_hbm.at[pl.ds((cbase + c) * aw, aw)],
                             auxb.at[pl.ds(slot * aw, aw)], sema)

        def wait_aux(slot):
            del slot
            pltpu.make_async_copy(aux_hbm.at[pl.ds(0, aw)],
                                  auxb.at[pl.ds(0, aw)], sema).wait()

        def fire_rows(c, slot):
            pltpu.async_copy(
                q_hbm.at[auxb.at[pl.ds(slot * aw, _CHUNK)]], qbuf.at[slot],
                semq)
            pltpu.async_copy(
                k_hbm.at[auxb.at[pl.ds(slot * aw + _CHUNK, _CHUNK)]],
                kbuf.at[slot], semk)

        def wait_rows(slot):
            pltpu.make_async_copy(q_hbm.at[pl.ds(0, _CHUNK)], qbuf.at[slot],
                                  semq).wait()
            pltpu.make_async_copy(k_hbm.at[pl.ds(0, _CHUNK)], kbuf.at[slot],
                                  semk).wait()

        pltpu.sync_copy(aux_hbm.at[pl.ds(cbase * aw, aw)],
                        auxb.at[pl.ds(0, aw)])
        fire_rows(0, 0)
        fire_aux(1, 1)
        lanes = lax.iota(jnp.int32, _L)

        def chunk_body(ci, carry):
            slot = lax.rem(ci, 2)
            nslot = 1 - slot
            wait_rows(slot)

            @pl.when(ci + 1 < nchunk)
            def _():
                wait_aux(nslot)
                fire_rows(ci + 1, nslot)

            perms = [lanes ^ s for s in (1, 2, 4, 8)]
            himask = jnp.full((_L,), jnp.int32(-65536), jnp.int32)
            for g in range(_CHUNK // _L):
                ev = jnp.zeros((_L,), jnp.float32)
                for i in range(_L):
                    ei_ = g * _L + i
                    acc = jnp.zeros((_L,), jnp.float32)
                    for j in range(d // (2 * _L)):
                        qw = qbuf[slot, ei_, pl.ds(j * _L, _L)]
                        kw = kbuf[slot, ei_, pl.ds(j * _L, _L)]
                        qhi = plsc.bitcast(qw, jnp.float32)
                        qlo = plsc.bitcast(qw << 16, jnp.float32)
                        khi = plsc.bitcast(kw, jnp.float32)
                        klo = plsc.bitcast(kw << 16, jnp.float32)
                        acc = acc + qhi * khi + qlo * klo
                    for pm in perms:
                        acc = acc + _shuffle(acc, pm)
                    ev = jnp.where(lanes == i, acc, ev)
                ew = plsc.bitcast(
                    auxb[pl.ds(slot * aw + 2 * _CHUNK + g * _L, _L)],
                    jnp.float32)
                zv[pl.ds(ci * _CHUNK + g * _L, _L)] = ev * ew * inv_scale

            @pl.when(ci + 2 < nchunk)
            def _():
                fire_aux(ci + 2, slot)

            return carry

        lax.fori_loop(0, nchunk, chunk_body, 0)
        pltpu.sync_copy(zv, z_hbm.at[pl.ds(wid * epw, epw)])

    return energy_kernel


def _softmax_body(z_ref, a_ref):
    z = z_ref[...]
    m = jnp.max(z)
    p = jnp.exp(z - m)
    a_ref[...] = p / jnp.sum(p)


def _softmax(z2d):
    return pl.pallas_call(
        _softmax_body,
        out_shape=jax.ShapeDtypeStruct(z2d.shape, jnp.float32),
    )(z2d)


def _make_scatter(n, e, d):
    epw = e // _NW
    nchunk = epw // _CHUNK
    zrows = 40
    ncopy_total = n // zrows
    ncopy_iters = -(-ncopy_total // _NS)
    orows = 200
    nout_total = n // orows
    nout_iters = -(-nout_total // _NS)
    mesh = plsc.VectorSubcoreMesh(core_axis_name="c", subcore_axis_name="s")

    @functools.partial(
        pl.kernel,
        out_type=jax.ShapeDtypeStruct((_NC, n, d), jnp.float32),
        mesh=mesh,
        scratch_types=[
            pltpu.VMEM((2 * 3 * _CHUNK,), jnp.int32),
            pltpu.VMEM((2 * _CHUNK,), jnp.float32),
            pltpu.VMEM((_CHUNK,), jnp.int32),
            pltpu.VMEM((2, _CHUNK, d), jnp.float32),
            pltpu.VMEM((zrows, d), jnp.float32),
            pltpu.VMEM_SHARED((n, d), jnp.float32),
            pltpu.SemaphoreType.DMA,
            pltpu.SemaphoreType.DMA,
            pltpu.SemaphoreType.DMA,
            pltpu.SemaphoreType.DMA,
        ],
        compiler_params=pltpu.CompilerParams(needs_layout_passes=False),
    )
    def scatter_kernel(v_hbm, aux_hbm, attn_hbm, out_hbm,
                       auxb, abuf, rc, vbuf, zbuf, acc_sh,
                       semv, sems, sema, semw):
        cid = lax.axis_index("c")
        sid = lax.axis_index("s")
        wid = sid * _NC + cid
        cbase = wid * nchunk
        ebase = wid * epw

        def zrow_body(r, carry):
            for j in range(d // _L):
                zbuf[r, pl.ds(j * _L, _L)] = jnp.zeros((_L,), jnp.float32)
            return carry

        lax.fori_loop(0, zrows, zrow_body, 0)

        def zcopy_body(t, carry):
            blk = t * _NS + sid

            @pl.when(blk < ncopy_total)
            def _():
                pltpu.sync_copy(zbuf, acc_sh.at[pl.ds(blk * zrows, zrows)])

            return carry

        lax.fori_loop(0, ncopy_iters, zcopy_body, 0)
        plsc.subcore_barrier()

        aw = 3 * _CHUNK

        def fire_aux(c, slot):
            pltpu.async_copy(aux_hbm.at[pl.ds((cbase + c) * aw, aw)],
                             auxb.at[pl.ds(slot * aw, aw)], sema)
            pltpu.async_copy(attn_hbm.at[pl.ds(ebase + c * _CHUNK, _CHUNK)],
                             abuf.at[pl.ds(slot * _CHUNK, _CHUNK)], semw)

        def wait_aux(slot):
            del slot
            pltpu.make_async_copy(aux_hbm.at[pl.ds(0, aw)],
                                  auxb.at[pl.ds(0, aw)], sema).wait()
            pltpu.make_async_copy(attn_hbm.at[pl.ds(0, _CHUNK)],
                                  abuf.at[pl.ds(0, _CHUNK)], semw).wait()

        def fire_v(c, slot):
            pltpu.async_copy(
                v_hbm.at[auxb.at[pl.ds(slot * aw + _CHUNK, _CHUNK)]],
                vbuf.at[slot], semv)

        def wait_v(slot):
            pltpu.make_async_copy(v_hbm.at[pl.ds(0, _CHUNK)], vbuf.at[slot],
                                  semv).wait()

        def drain_scatter(slot):
            pltpu.make_async_copy(vbuf.at[slot], acc_sh.at[rc], sems).wait()

        pltpu.sync_copy(aux_hbm.at[pl.ds(cbase * aw, aw)],
                        auxb.at[pl.ds(0, aw)])
        pltpu.sync_copy(attn_hbm.at[pl.ds(ebase, _CHUNK)],
                        abuf.at[pl.ds(0, _CHUNK)])
        fire_v(0, 0)
        fire_aux(1, 1)

        def chunk_body(ci, carry):
            slot = lax.rem(ci, 2)
            nslot = 1 - slot
            wait_v(slot)

            @pl.when(ci > 0)
            def _():
                drain_scatter(nslot)

            @pl.when(ci + 1 < nchunk)
            def _():
                wait_aux(nslot)
                fire_v(ci + 1, nslot)

            vc = vbuf.at[slot]
            for g in range(_CHUNK // _L):
                off = g * _L
                rc[pl.ds(off, _L)] = auxb[pl.ds(slot * aw + off, _L)]
                a16 = abuf[pl.ds(slot * _CHUNK + off, _L)]
                for i in range(_L):
                    ei = off + i
                    a = a16[i]
                    for j in range(d // _L):
                        vc[ei, pl.ds(j * _L, _L)] = (
                            vc[ei, pl.ds(j * _L, _L)] * a)
            pltpu.async_copy(vbuf.at[slot], acc_sh.at[rc], sems, add=True)

            @pl.when(ci + 2 < nchunk)
            def _():
                fire_aux(ci + 2, slot)

            return carry

        lax.fori_loop(0, nchunk, chunk_body, 0)
        drain_scatter((nchunk - 1) % 2)
        plsc.subcore_barrier()

        def ocopy_body(t, carry):
            blk = t * _NS + sid

            @pl.when(blk < nout_total)
            def _():
                r0 = blk * orows
                pltpu.sync_copy(acc_sh.at[pl.ds(r0, orows)],
                                out_hbm.at[cid, pl.ds(r0, orows)])

            return carry

        lax.fori_loop(0, nout_iters, ocopy_body, 0)

    return scatter_kernel


def _combine_body(p_ref, o_ref):
    o_ref[...] = p_ref[0] + p_ref[1]


def _combine(part):
    _, n, d = part.shape
    blk = 2000
    return pl.pallas_call(
        _combine_body,
        grid=(n // blk,),
        in_specs=[pl.BlockSpec((2, blk, d), lambda i: (0, i, 0))],
        out_specs=pl.BlockSpec((blk, d), lambda i: (i, 0)),
        out_shape=jax.ShapeDtypeStruct((n, d), jnp.float32),
    )(part)


def kernel(x, edge_index, edge_weight, Wq, bq, Wk, bk, Wv, bv):
    n, d = x.shape
    e = edge_weight.shape[0]

    ew_bits = lax.bitcast_convert_type(edge_weight, jnp.int32)
    aux = jnp.stack([edge_index[0], edge_index[1], ew_bits], axis=0)
    aux3 = aux.reshape(3, e // _CHUNK, _CHUNK).transpose(1, 0, 2)
    aux3 = aux3.reshape(e // _CHUNK * 3 * _CHUNK)

    q, k, v = _qkv(x, Wq, Wk, Wv, bq, bk, bv)
    qp = lax.bitcast_convert_type(q.reshape(n, d // 2, 2), jnp.int32)
    kp = lax.bitcast_convert_type(k.reshape(n, d // 2, 2), jnp.int32)
    z = _make_energy(n, e, d)(qp, kp, aux3)
    attn = _softmax(z.reshape(e // 128, 128)).reshape(e)
    part = _make_scatter(n, e, d)(v, aux3, attn)
    return _combine(part)
